# Initial kernel scaffold; baseline (speedup 1.0000x reference)
import jax
import jax.numpy as jnp
from jax import lax
from jax.experimental import pallas as pl
from jax.experimental.pallas import tpu as pltpu
from jax.experimental.pallas import tpu_sc as plsc

T, H, W = 16, 32, 32
TT, HH, WW = 8, 16, 16
C = 128
B = 8
V = 65536
NTOK = TT * HH * WW
NROWS = NTOK * B * 8

NC, NS, L = 2, 16, 16
NW = NC * NS
ROWS_PER_W = NROWS // NW
CHUNK = 128
NCHUNK = ROWS_PER_W // CHUNK
XROWS = 256

_mesh = plsc.VectorSubcoreMesh(
    core_axis_name="c", subcore_axis_name="s", num_cores=NC, num_subcores=NS
)


def _body(x_hbm, table_hbm, out_hbm, xv, idx, rows, sem0, sem1):
    wid = lax.axis_index("s") * NC + lax.axis_index("c")
    tt = wid >> 2
    q = wid & 3

    for gt in range(2):
        src_row = (2 * tt + gt) * (H * W) + q * XROWS
        pltpu.sync_copy(x_hbm.at[pl.ds(src_row, XROWS)], xv.at[gt])

    iota = lax.iota(jnp.int32, L)

    @pl.loop(0, ROWS_PER_W // L)
    def _compute_idx(j0):
        j = j0 * L + iota
        g = j & 7
        b = (j >> 3) & 7
        ww = (j >> 6) & 15
        dhh = (j >> 10) & 3
        gt = g & 1
        gh = (g >> 1) & 1
        gw = (g >> 2) & 1
        i1 = (2 * dhh + gh) * 32 + (2 * ww + gw)
        vals = plsc.load_gather(xv, [gt, i1, b])
        idx[j0 >> 3, pl.ds((j0 & 7) * L, L)] = vals

    out_base = wid * ROWS_PER_W
    sems = (sem0, sem1)

    def _gather(c, s):
        pltpu.make_async_copy(table_hbm.at[idx.at[c]], rows.at[s], sems[s]).start()

    _gather(0, 0)

    @pl.loop(0, NCHUNK, step=2)
    def _chunk_loop(c0):
        for s in range(2):
            c = c0 + s

            @pl.when(c + 1 < NCHUNK)
            def _():
                _gather(c + 1, 1 - s)

            pltpu.make_async_copy(
                table_hbm.at[idx.at[c]], rows.at[s], sems[s]
            ).wait()
            pltpu.sync_copy(
                rows.at[s], out_hbm.at[pl.ds(out_base + c * CHUNK, CHUNK)]
            )


_lookup = pl.kernel(
    _body,
    out_type=jax.ShapeDtypeStruct((NROWS, C), jnp.float32),
    mesh=_mesh,
    scratch_types=[
        pltpu.VMEM((2, XROWS, B), jnp.int32),
        pltpu.VMEM((NCHUNK, CHUNK), jnp.int32),
        pltpu.VMEM((2, CHUNK, C), jnp.float32),
        pltpu.SemaphoreType.DMA,
        pltpu.SemaphoreType.DMA,
    ],
)


@jax.jit
def kernel(x, table):
    out = _lookup(x, table)
    return out.reshape(NTOK, B, 8 * C)


# speedup vs baseline: 5.2149x; 5.2149x over previous
import jax
import jax.numpy as jnp
from jax import lax
from jax.experimental import pallas as pl
from jax.experimental.pallas import tpu as pltpu
from jax.experimental.pallas import tpu_sc as plsc

T, H, W = 16, 32, 32
C = 128
B = 8
NTOK = (T // 2) * (H // 2) * (W // 2)
NROWS = NTOK * B * 8

NC, NS, L = 2, 16, 16
NW = NC * NS
EL_PER_W = T * H * W * B // NW
CHUNK = 128
NCHUNK = EL_PER_W // CHUNK
NBUF = 4
LOOKAHEAD = 2

_mesh = plsc.VectorSubcoreMesh(
    core_axis_name="c", subcore_axis_name="s", num_cores=NC, num_subcores=NS
)


def _body(x_hbm, table_hbm, out_hbm, xin, oidx, rows, *sems):
    gsem = sems[:NBUF]
    ssem = sems[NBUF:]
    wid = lax.axis_index("s") * NC + lax.axis_index("c")

    pltpu.sync_copy(x_hbm.at[pl.ds(wid * NCHUNK, NCHUNK)], xin)

    tt = wid >> 2
    gt = (wid >> 1) & 1
    hh0 = 8 * (wid & 1)
    wbase = tt * 16384 + gt
    iota = lax.iota(jnp.int32, L)

    @pl.loop(0, EL_PER_W // L)
    def _compute_oidx(j0):
        c = j0 >> 3
        base = (
            wbase
            + (hh0 + (c >> 2)) * 1024
            + ((c >> 1) & 1) * 2
            + (c & 1) * 512
        )
        r = (j0 & 7) * L + iota
        pat = ((r >> 4) * 64) + ((r & 7) * 8) + (((r >> 3) & 1) * 4)
        oidx[c, pl.ds((j0 & 7) * L, L)] = base + pat

    def _gather(c, s):
        pltpu.make_async_copy(table_hbm.at[xin.at[c]], rows.at[s], gsem[s]).start()

    def _scatter(c, s):
        pltpu.make_async_copy(rows.at[s], out_hbm.at[oidx.at[c]], ssem[s]).start()

    def _wait_gather(c, s):
        pltpu.make_async_copy(table_hbm.at[xin.at[c]], rows.at[s], gsem[s]).wait()

    def _wait_scatter(c, s):
        pltpu.make_async_copy(rows.at[s], out_hbm.at[oidx.at[c]], ssem[s]).wait()

    for s in range(LOOKAHEAD):
        _gather(s, s)

    @pl.loop(0, NCHUNK, step=NBUF)
    def _chunk_loop(c0):
        for s in range(NBUF):
            c = c0 + s
            _wait_gather(c, s)
            _scatter(c, s)
            s2 = (s + LOOKAHEAD) % NBUF
            cn = c + LOOKAHEAD

            @pl.when(cn >= NBUF)
            def _():
                _wait_scatter(cn - NBUF, s2)

            @pl.when(cn < NCHUNK)
            def _():
                _gather(cn, s2)

    for c in range(NCHUNK - NBUF + LOOKAHEAD, NCHUNK):
        _wait_scatter(c, c % NBUF)


_lookup = pl.kernel(
    _body,
    out_type=jax.ShapeDtypeStruct((NROWS, C), jnp.float32),
    mesh=_mesh,
    scratch_types=[
        pltpu.VMEM((NCHUNK, CHUNK), jnp.int32),
        pltpu.VMEM((NCHUNK, CHUNK), jnp.int32),
        pltpu.VMEM((NBUF, CHUNK, C), jnp.float32),
    ]
    + [pltpu.SemaphoreType.DMA] * (2 * NBUF),
)


@jax.jit
def kernel(x, table):
    out = _lookup(x.reshape(T * H * W * B // CHUNK, CHUNK), table)
    return out.reshape(NTOK, B, 8 * C)
